# Initial kernel scaffold; baseline (speedup 1.0000x reference)
import jax
import jax.numpy as jnp
from jax.experimental import pallas as pl


def kernel(inputs, token_table, position_table):
    raise NotImplementedError("write your pallas kernel here")


# speedup vs baseline: 3.1085x; 3.1085x over previous
import functools

import jax
import jax.numpy as jnp
from jax import lax
from jax.experimental import pallas as pl
from jax.experimental.pallas import tpu as pltpu
from jax.experimental.pallas import tpu_sc as plsc

SEQ = 200
D = 64
NUM_CORES = 2
NUM_SUBCORES = 16
NUM_WORKERS = NUM_CORES * NUM_SUBCORES
LANES = 16
G0, G1 = 128, 72


def kernel(inputs, token_table, position_table):
    batch, seq = inputs.shape
    vocab, d = token_table.shape
    total = batch * seq
    rows_per_w = total // NUM_WORKERS
    seq_per_w = rows_per_w // seq

    idx_flat = inputs.reshape(total).astype(jnp.int32)

    mesh = plsc.VectorSubcoreMesh(core_axis_name="c", subcore_axis_name="s")

    @functools.partial(
        pl.kernel,
        out_type=jax.ShapeDtypeStruct((total, d), jnp.float32),
        mesh=mesh,
        scratch_types=[
            pltpu.VMEM((SEQ,), jnp.int32),
            pltpu.VMEM((SEQ, D), jnp.float32),
            pltpu.VMEM((SEQ, D), jnp.float32),
            pltpu.SemaphoreType.DMA,
        ],
        compiler_params=pltpu.CompilerParams(use_tc_tiling_on_sc=False),
    )
    def sc_embed(idx_hbm, tab_hbm, pos_hbm, out_hbm, idx_v, rows_v, pos_v, sem):
        wid = lax.axis_index("s") * NUM_CORES + lax.axis_index("c")
        base = wid * rows_per_w
        pltpu.sync_copy(pos_hbm, pos_v)

        @pl.loop(0, seq_per_w)
        def _(s):
            off = base + s * SEQ
            pltpu.sync_copy(idx_hbm.at[pl.ds(off, SEQ)], idx_v)
            c0 = pltpu.async_copy(
                tab_hbm.at[idx_v.at[pl.ds(0, G0)]], rows_v.at[pl.ds(0, G0)], sem
            )
            c1 = pltpu.async_copy(
                tab_hbm.at[idx_v.at[pl.ds(G0, G1)]], rows_v.at[pl.ds(G0, G1)], sem
            )
            c0.wait()
            c1.wait()

            @pl.loop(0, SEQ)
            def _(r):
                for j in range(D // LANES):
                    sl = (pl.ds(r, 1), pl.ds(j * LANES, LANES))
                    rows_v.at[sl][...] = rows_v.at[sl][...] + pos_v.at[sl][...]

            pltpu.sync_copy(rows_v, out_hbm.at[pl.ds(off, SEQ)])

    out = sc_embed(idx_flat, token_table, position_table)
    return out.reshape(batch, seq, d)


# speedup vs baseline: 3.9840x; 1.2816x over previous
import functools

import jax
import jax.numpy as jnp
from jax import lax
from jax.experimental import pallas as pl
from jax.experimental.pallas import tpu as pltpu
from jax.experimental.pallas import tpu_sc as plsc

SEQ = 200
D = 64
NUM_CORES = 2
NUM_SUBCORES = 16
NUM_WORKERS = NUM_CORES * NUM_SUBCORES
LANES = 16
G0, G1 = 128, 72


def kernel(inputs, token_table, position_table):
    batch, seq = inputs.shape
    vocab, d = token_table.shape
    total = batch * seq
    rows_per_w = total // NUM_WORKERS
    seq_per_w = rows_per_w // seq

    idx_flat = inputs.reshape(total).astype(jnp.int32)

    mesh = plsc.VectorSubcoreMesh(core_axis_name="c", subcore_axis_name="s")

    @functools.partial(
        pl.kernel,
        out_type=jax.ShapeDtypeStruct((total, d), jnp.float32),
        mesh=mesh,
        scratch_types=[
            pltpu.VMEM((rows_per_w,), jnp.int32),
            pltpu.VMEM((SEQ, D), jnp.float32),
            pltpu.VMEM((SEQ, D), jnp.float32),
            pltpu.VMEM((SEQ, D), jnp.float32),
            pltpu.SemaphoreType.DMA,
            pltpu.SemaphoreType.DMA,
            pltpu.SemaphoreType.DMA,
            pltpu.SemaphoreType.DMA,
        ],
        compiler_params=pltpu.CompilerParams(use_tc_tiling_on_sc=False),
    )
    def sc_embed(idx_hbm, tab_hbm, pos_hbm, out_hbm, idx_v, pos_v, rows0,
                 rows1, sem_g0, sem_g1, sem_o0, sem_o1):
        wid = lax.axis_index("s") * NUM_CORES + lax.axis_index("c")
        base = wid * rows_per_w

        def issue_gather(s, rows_b, sem):
            o = s * SEQ
            pltpu.async_copy(
                tab_hbm.at[idx_v.at[pl.ds(o, G0)]], rows_b.at[pl.ds(0, G0)], sem
            )
            pltpu.async_copy(
                tab_hbm.at[idx_v.at[pl.ds(o + G0, G1)]],
                rows_b.at[pl.ds(G0, G1)], sem,
            )

        def wait_gather(s, rows_b, sem):
            o = s * SEQ
            pltpu.make_async_copy(
                tab_hbm.at[idx_v.at[pl.ds(o, G0)]], rows_b.at[pl.ds(0, G0)], sem
            ).wait()
            pltpu.make_async_copy(
                tab_hbm.at[idx_v.at[pl.ds(o + G0, G1)]],
                rows_b.at[pl.ds(G0, G1)], sem,
            ).wait()

        def issue_out(s, rows_b, sem):
            pltpu.async_copy(rows_b, out_hbm.at[pl.ds(base + s * SEQ, SEQ)], sem)

        def wait_out(s, rows_b, sem):
            pltpu.make_async_copy(
                rows_b, out_hbm.at[pl.ds(base + s * SEQ, SEQ)], sem
            ).wait()

        def add_pos(rows_b):
            @pl.loop(0, SEQ)
            def _(r):
                for j in range(D // LANES):
                    sl = (pl.ds(r, 1), pl.ds(j * LANES, LANES))
                    plsc.addupdate(rows_b.at[sl], pos_v.at[sl][...])

        pltpu.sync_copy(idx_hbm.at[pl.ds(base, rows_per_w)], idx_v)
        pltpu.sync_copy(pos_hbm, pos_v)

        issue_gather(0, rows0, sem_g0)
        issue_gather(1, rows1, sem_g1)
        wait_gather(0, rows0, sem_g0)
        add_pos(rows0)
        issue_out(0, rows0, sem_o0)

        @pl.loop(0, (seq_per_w - 2) // 2)
        def _(i):
            s1 = 2 * i + 1
            wait_out(s1 - 1, rows0, sem_o0)
            issue_gather(s1 + 1, rows0, sem_g0)
            wait_gather(s1, rows1, sem_g1)
            add_pos(rows1)
            issue_out(s1, rows1, sem_o1)

            s2 = 2 * i + 2
            wait_out(s2 - 1, rows1, sem_o1)
            issue_gather(s2 + 1, rows1, sem_g1)
            wait_gather(s2, rows0, sem_g0)
            add_pos(rows0)
            issue_out(s2, rows0, sem_o0)

        last = seq_per_w - 1
        wait_out(last - 1, rows0, sem_o0)
        wait_gather(last, rows1, sem_g1)
        add_pos(rows1)
        issue_out(last, rows1, sem_o1)
        wait_out(last, rows1, sem_o1)

    out = sc_embed(idx_flat, token_table, position_table)
    return out.reshape(batch, seq, d)
